# Initial kernel scaffold; baseline (speedup 1.0000x reference)
import functools

import jax
import jax.numpy as jnp
from jax import lax
from jax.experimental import pallas as pl
from jax.experimental.pallas import tpu as pltpu
from jax.experimental.pallas import tpu_sc as plsc

N = 10000
E = 320000
F = 128
C = 40
CP = 64
NPAD = 10240
SUB = 125
NSUBROWS = E // SUB
TSUB_PROP = (E // 2) // 16 // SUB
TSUB_DEG = E // 16 // SUB

_MESH = plsc.VectorSubcoreMesh(core_axis_name="c", subcore_axis_name="s")


def _quake_rsqrt(d):
    i = plsc.bitcast(d, jnp.int32)
    i = jnp.int32(0x5F3759DF) - (i >> 1)
    y = plsc.bitcast(i, jnp.float32)
    for _ in range(3):
        y = y * (jnp.float32(1.5) - jnp.float32(0.5) * d * y * y)
    return y


@functools.partial(
    pl.kernel,
    out_type=jax.ShapeDtypeStruct((NPAD,), jnp.float32),
    mesh=_MESH,
    scratch_types=[
        pltpu.VMEM_SHARED((NPAD,), jnp.float32),
        pltpu.VMEM((TSUB_DEG, SUB), jnp.int32),
        pltpu.VMEM((SUB,), jnp.float32),
        pltpu.VMEM((640,), jnp.float32),
        pltpu.SemaphoreType.DMA,
        pltpu.SemaphoreType.DMA,
    ],
)
def _deg_dinv_kernel(e3_hbm, dinv_hbm, deg_sh, idx_v, ones_v, stage_v, s_sem0, s_sem1):
    cid = lax.axis_index("c")
    sid = lax.axis_index("s")

    def store16(ref, n, val):
        def body(j, _):
            ref[pl.ds(j * 16, 16)] = jnp.full((16,), val, jnp.float32)
            return 0
        lax.fori_loop(0, n // 16, body, 0)

    store16(ones_v, SUB - 5, 1.0)
    ones_v[pl.ds(SUB - 16, 16)] = jnp.full((16,), 1.0, jnp.float32)
    store16(stage_v, 640, 1.0)
    pltpu.sync_copy(stage_v, deg_sh.at[pl.ds(sid * 640, 640)])
    pltpu.sync_copy(e3_hbm.at[1, pl.ds(sid * TSUB_DEG, TSUB_DEG)], idx_v)
    plsc.subcore_barrier()

    SPS = 4
    NSUP = TSUB_DEG // SPS

    def fire(t, sem):
        for k in range(SPS):
            pltpu.async_copy(ones_v, deg_sh.at[idx_v.at[t * SPS + k]], sem,
                             add=True)

    def drain(t, sem):
        for k in range(SPS):
            pltpu.make_async_copy(ones_v, deg_sh.at[idx_v.at[t * SPS + k]],
                                  sem).wait()

    fire(0, s_sem0)
    fire(1, s_sem1)

    def body(i, _):
        t = 2 * i
        drain(t - 2, s_sem0)
        fire(t, s_sem0)
        drain(t - 1, s_sem1)
        fire(t + 1, s_sem1)
        return 0

    lax.fori_loop(1, NSUP // 2, body, 0)
    drain(NSUP - 2, s_sem0)
    drain(NSUP - 1, s_sem1)
    plsc.subcore_barrier()

    wid = sid * 2 + cid
    base = wid * 320
    pltpu.sync_copy(deg_sh.at[pl.ds(base, 320)], stage_v.at[pl.ds(0, 320)])

    def rbody(j, _):
        d = stage_v[pl.ds(j * 16, 16)]
        stage_v[pl.ds(320 + j * 16, 16)] = _quake_rsqrt(d)
        return 0

    lax.fori_loop(0, 20, rbody, 0)
    pltpu.sync_copy(stage_v.at[pl.ds(320, 320)], dinv_hbm.at[pl.ds(base, 320)])


def _make_prop_kernel(d):
    sps = 2 if d == F else 4
    nsup = TSUB_PROP // sps
    rows_per_tile = N // 16

    @functools.partial(
        pl.kernel,
        out_type=jax.ShapeDtypeStruct((2, N, d), jnp.float32),
        mesh=_MESH,
        scratch_types=[
            pltpu.VMEM_SHARED((N, d), jnp.float32),
            pltpu.VMEM((2, TSUB_PROP, SUB), jnp.int32),
            pltpu.VMEM((2, sps, SUB, d), jnp.float32),
            pltpu.SemaphoreType.DMA,
            pltpu.SemaphoreType.DMA,
            pltpu.SemaphoreType.DMA,
        ],
    )
    def prop(e3_hbm, xp_hbm, hxp_hbm, out_hbm, acc_sh, idx_v, rows_v,
             g_sem, s_sem0, s_sem1):
        cid = lax.axis_index("c")
        sid = lax.axis_index("s")
        rbase = sid * rows_per_tile
        subbase = cid * (NSUBROWS // 2) + sid * TSUB_PROP
        pltpu.sync_copy(e3_hbm.at[:, pl.ds(subbase, TSUB_PROP)], idx_v)
        pltpu.sync_copy(hxp_hbm.at[pl.ds(rbase, rows_per_tile)],
                        acc_sh.at[pl.ds(rbase, rows_per_tile)])
        plsc.subcore_barrier()

        def fire(t, b, sem):
            for k in range(sps):
                pltpu.async_copy(xp_hbm.at[idx_v.at[0, t * sps + k]],
                                 rows_v.at[b, k], g_sem)
            for k in range(sps):
                pltpu.make_async_copy(xp_hbm.at[idx_v.at[0, t * sps + k]],
                                      rows_v.at[b, k], g_sem).wait()
            for k in range(sps):
                pltpu.async_copy(rows_v.at[b, k],
                                 acc_sh.at[idx_v.at[1, t * sps + k]], sem,
                                 add=True)

        def drain(t, b, sem):
            for k in range(sps):
                pltpu.make_async_copy(rows_v.at[b, k],
                                      acc_sh.at[idx_v.at[1, t * sps + k]],
                                      sem).wait()

        fire(0, 0, s_sem0)
        fire(1, 1, s_sem1)

        def body(i, _):
            t = 2 * i
            drain(t - 2, 0, s_sem0)
            fire(t, 0, s_sem0)
            drain(t - 1, 1, s_sem1)
            fire(t + 1, 1, s_sem1)
            return 0

        lax.fori_loop(1, nsup // 2, body, 0)
        drain(nsup - 2, 0, s_sem0)
        drain(nsup - 1, 1, s_sem1)
        plsc.subcore_barrier()
        pltpu.sync_copy(acc_sh.at[pl.ds(rbase, rows_per_tile)],
                        out_hbm.at[cid, pl.ds(rbase, rows_per_tile)])

    return prop


_prop128 = _make_prop_kernel(F)
_prop64 = _make_prop_kernel(CP)


_BLK = 1000


def _scale_body(dinv_ref, x_ref, xp_ref, hxp_ref):
    xp = dinv_ref[...] * x_ref[...]
    xp_ref[...] = xp
    hxp_ref[...] = 0.5 * xp


def _mlp_body(dinv_ref, pa_ref, pb_ref, w1_ref, b1_ref, w2_ref, yp_ref, hyp_ref):
    g = dinv_ref[...] * (pa_ref[...] + pb_ref[...])
    h = jnp.maximum(
        jnp.dot(g, w1_ref[...], preferred_element_type=jnp.float32)
        + b1_ref[...], 0.0)
    y = jnp.dot(h, w2_ref[...], preferred_element_type=jnp.float32)
    yp = dinv_ref[...] * y
    yp_ref[...] = yp
    hyp_ref[...] = 0.5 * yp


def _lsm_body(dinv_ref, pa_ref, pb_ref, b2_ref, out_ref):
    logits = (dinv_ref[...] * (pa_ref[...] + pb_ref[...]))[:, :C] + b2_ref[...]
    m = jnp.max(logits, axis=1, keepdims=True)
    z = logits - m
    lse = jnp.log(jnp.sum(jnp.exp(z), axis=1, keepdims=True))
    out_ref[...] = z - lse


def _row_spec(w):
    return pl.BlockSpec((_BLK, w), lambda i: (i, 0))


def _full_spec(h, w):
    return pl.BlockSpec((h, w), lambda i: (0, 0))


def kernel(x, edge_index, W1, b1, W2, b2):
    e3 = edge_index.reshape(2, NSUBROWS, SUB)
    dinv = _deg_dinv_kernel(e3)
    dinv2d = dinv[:N].reshape(N, 1)

    xp, hxp = pl.pallas_call(
        _scale_body,
        grid=(N // _BLK,),
        in_specs=[_row_spec(1), _row_spec(F)],
        out_specs=[_row_spec(F), _row_spec(F)],
        out_shape=[jax.ShapeDtypeStruct((N, F), jnp.float32)] * 2,
    )(dinv2d, x)

    p1 = _prop128(e3, xp, hxp)

    W2p = jnp.pad(W2, ((0, 0), (0, CP - C)))
    yp, hyp = pl.pallas_call(
        _mlp_body,
        grid=(N // _BLK,),
        in_specs=[_row_spec(1), _row_spec(F), _row_spec(F),
                  _full_spec(F, F), _full_spec(1, F), _full_spec(F, CP)],
        out_specs=[_row_spec(CP), _row_spec(CP)],
        out_shape=[jax.ShapeDtypeStruct((N, CP), jnp.float32)] * 2,
    )(dinv2d, p1[0], p1[1], W1, b1.reshape(1, F), W2p)

    p2 = _prop64(e3, yp, hyp)

    out = pl.pallas_call(
        _lsm_body,
        grid=(N // _BLK,),
        in_specs=[_row_spec(1), _row_spec(CP), _row_spec(CP),
                  _full_spec(1, C)],
        out_specs=_row_spec(C),
        out_shape=jax.ShapeDtypeStruct((N, C), jnp.float32),
    )(dinv2d, p2[0], p2[1], b2.reshape(1, C))
    return out


# speedup vs baseline: 29.2203x; 29.2203x over previous
import functools

import jax
import jax.numpy as jnp
from jax import lax
from jax.experimental import pallas as pl
from jax.experimental.pallas import tpu as pltpu
from jax.experimental.pallas import tpu_sc as plsc

N = 10000
E = 320000
F = 128
C = 40
DH = 64
NPAD = 10240
SUB = 125
NSUBROWS = E // SUB
TSUB_PROP = (E // 2) // 16 // SUB
TSUB_DEG = E // 16 // SUB
RPT = NPAD // 16

_MESH = plsc.VectorSubcoreMesh(core_axis_name="c", subcore_axis_name="s")


@functools.partial(
    pl.kernel,
    out_type=jax.ShapeDtypeStruct((NPAD,), jnp.float32),
    mesh=_MESH,
    scratch_types=[
        pltpu.VMEM_SHARED((NPAD,), jnp.float32),
        pltpu.VMEM((TSUB_DEG, SUB), jnp.int32),
        pltpu.VMEM((SUB,), jnp.float32),
        pltpu.VMEM((640,), jnp.float32),
        pltpu.SemaphoreType.DMA,
        pltpu.SemaphoreType.DMA,
    ],
)
def _deg_kernel(e3_hbm, deg_hbm, deg_sh, idx_v, ones_v, stage_v, s_sem0, s_sem1):
    cid = lax.axis_index("c")
    sid = lax.axis_index("s")

    def store16(ref, n, val):
        def body(j, _):
            ref[pl.ds(j * 16, 16)] = jnp.full((16,), val, jnp.float32)
            return 0
        lax.fori_loop(0, n // 16, body, 0)

    store16(ones_v, 112, 1.0)
    ones_v[pl.ds(SUB - 16, 16)] = jnp.full((16,), 1.0, jnp.float32)
    store16(stage_v, 640, 1.0)
    pltpu.sync_copy(stage_v, deg_sh.at[pl.ds(sid * 640, 640)])
    pltpu.sync_copy(e3_hbm.at[1, pl.ds(sid * TSUB_DEG, TSUB_DEG)], idx_v)
    plsc.subcore_barrier()

    SPS = 4
    NSUP = TSUB_DEG // SPS

    def fire(t, sem):
        for k in range(SPS):
            pltpu.async_copy(ones_v, deg_sh.at[idx_v.at[t * SPS + k]], sem,
                             add=True)

    def drain(t, sem):
        for k in range(SPS):
            pltpu.make_async_copy(ones_v, deg_sh.at[idx_v.at[t * SPS + k]],
                                  sem).wait()

    fire(0, s_sem0)
    fire(1, s_sem1)

    def body(i, _):
        t = 2 * i
        drain(t - 2, s_sem0)
        fire(t, s_sem0)
        drain(t - 1, s_sem1)
        fire(t + 1, s_sem1)
        return 0

    lax.fori_loop(1, NSUP // 2, body, 0)
    drain(NSUP - 2, s_sem0)
    drain(NSUP - 1, s_sem1)
    plsc.subcore_barrier()

    base = (sid * 2 + cid) * 320
    pltpu.sync_copy(deg_sh.at[pl.ds(base, 320)], stage_v.at[pl.ds(0, 320)])
    pltpu.sync_copy(stage_v.at[pl.ds(0, 320)], deg_hbm.at[pl.ds(base, 320)])


SPS = 2
NSUP = TSUB_PROP // SPS


@functools.partial(
    pl.kernel,
    out_type=jax.ShapeDtypeStruct((2, NPAD, DH), jnp.float32),
    mesh=_MESH,
    compiler_params=pltpu.CompilerParams(use_tc_tiling_on_sc=False),
    scratch_types=[
        pltpu.VMEM_SHARED((NPAD, DH), jnp.float32),
        pltpu.VMEM((2, TSUB_PROP, SUB), jnp.int32),
        pltpu.VMEM((2, SPS, 128, DH), jnp.float32),
        pltpu.SemaphoreType.DMA,
        pltpu.SemaphoreType.DMA,
        pltpu.SemaphoreType.DMA,
    ],
)
def _prop_kernel(e3_hbm, xp_hbm, out_hbm, acc_sh, idx_v, rows_v,
                 g_sem, s_sem0, s_sem1):
    cid = lax.axis_index("c")
    sid = lax.axis_index("s")
    rbase = sid * RPT
    subbase = cid * (NSUBROWS // 2) + sid * TSUB_PROP
    pltpu.sync_copy(e3_hbm.at[:, pl.ds(subbase, TSUB_PROP)], idx_v)

    zb = rows_v.at[0, 0]

    def zbody(t, _):
        zb[t // 4, pl.ds((t % 4) * 16, 16)] = jnp.zeros((16,), jnp.float32)
        return 0

    lax.fori_loop(0, 512, zbody, 0)
    for j in range(RPT // 128):
        pltpu.sync_copy(zb, acc_sh.at[pl.ds(rbase + j * 128, 128)])
    plsc.subcore_barrier()

    def fire(t, b, sem):
        for k in range(SPS):
            pltpu.async_copy(xp_hbm.at[idx_v.at[0, t * SPS + k]],
                             rows_v.at[b, k, pl.ds(0, SUB)], g_sem)
        for k in range(SPS):
            pltpu.make_async_copy(xp_hbm.at[idx_v.at[0, t * SPS + k]],
                                  rows_v.at[b, k, pl.ds(0, SUB)], g_sem).wait()
        for k in range(SPS):
            pltpu.async_copy(rows_v.at[b, k, pl.ds(0, SUB)],
                             acc_sh.at[idx_v.at[1, t * SPS + k]], sem,
                             add=True)

    def drain(t, b, sem):
        for k in range(SPS):
            pltpu.make_async_copy(rows_v.at[b, k, pl.ds(0, SUB)],
                                  acc_sh.at[idx_v.at[1, t * SPS + k]],
                                  sem).wait()

    fire(0, 0, s_sem0)
    fire(1, 1, s_sem1)

    def body(i, _):
        t = 2 * i
        drain(t - 2, 0, s_sem0)
        fire(t, 0, s_sem0)
        drain(t - 1, 1, s_sem1)
        fire(t + 1, 1, s_sem1)
        return 0

    lax.fori_loop(1, NSUP // 2, body, 0)
    drain(NSUP - 2, 0, s_sem0)
    drain(NSUP - 1, 1, s_sem1)
    plsc.subcore_barrier()

    for j in range(RPT // 128):
        pltpu.sync_copy(acc_sh.at[pl.ds(rbase + j * 128, 128)], zb)
        pltpu.sync_copy(zb, out_hbm.at[cid, pl.ds(rbase + j * 128, 128)])


_BLK = 1024


def _scale_body(deg_ref, x_ref, dinv_ref, xp0_ref, xp1_ref):
    dinv = jax.lax.rsqrt(deg_ref[...])
    dinv_ref[...] = dinv
    xp = dinv * x_ref[...]
    xp0_ref[...] = xp[:, :DH]
    xp1_ref[...] = xp[:, DH:]


def _mlp_body(dinv_ref, pa0_ref, pb0_ref, pa1_ref, pb1_ref, xp0_ref, xp1_ref,
              w1_ref, b1_ref, w2_ref, yp_ref):
    dinv = dinv_ref[...]
    g0 = dinv * (pa0_ref[...] + pb0_ref[...] + xp0_ref[...])
    g1 = dinv * (pa1_ref[...] + pb1_ref[...] + xp1_ref[...])
    g = jnp.concatenate([g0, g1], axis=1)
    h = jnp.maximum(
        jnp.dot(g, w1_ref[...], preferred_element_type=jnp.float32)
        + b1_ref[...], 0.0)
    y = jnp.dot(h, w2_ref[...], preferred_element_type=jnp.float32)
    yp_ref[...] = dinv * y


def _lsm_body(dinv_ref, pa_ref, pb_ref, yp_ref, b2_ref, out_ref):
    logits = (dinv_ref[...] * (pa_ref[...] + pb_ref[...] + yp_ref[...]))[:, :C]
    logits = logits + b2_ref[...]
    m = jnp.max(logits, axis=1, keepdims=True)
    z = logits - m
    lse = jnp.log(jnp.sum(jnp.exp(z), axis=1, keepdims=True))
    out_ref[...] = z - lse


def _row_spec(w):
    return pl.BlockSpec((_BLK, w), lambda i: (i, 0))


def _full_spec(h, w):
    return pl.BlockSpec((h, w), lambda i: (0, 0))


def kernel(x, edge_index, W1, b1, W2, b2):
    e3 = edge_index.reshape(2, NSUBROWS, SUB)
    deg = _deg_kernel(e3)
    deg2d = deg.reshape(NPAD, 1)
    xpad = jnp.pad(x, ((0, NPAD - N), (0, 0)))

    dinv2d, xp0, xp1 = pl.pallas_call(
        _scale_body,
        grid=(NPAD // _BLK,),
        in_specs=[_row_spec(1), _row_spec(F)],
        out_specs=[_row_spec(1), _row_spec(DH), _row_spec(DH)],
        out_shape=[jax.ShapeDtypeStruct((NPAD, 1), jnp.float32),
                   jax.ShapeDtypeStruct((NPAD, DH), jnp.float32),
                   jax.ShapeDtypeStruct((NPAD, DH), jnp.float32)],
    )(deg2d, xpad)

    p0 = _prop_kernel(e3, xp0)
    p1 = _prop_kernel(e3, xp1)

    W2p = jnp.pad(W2, ((0, 0), (0, DH - C)))
    yp = pl.pallas_call(
        _mlp_body,
        grid=(NPAD // _BLK,),
        in_specs=[_row_spec(1)] + [_row_spec(DH)] * 6
        + [_full_spec(F, F), _full_spec(1, F), _full_spec(F, DH)],
        out_specs=_row_spec(DH),
        out_shape=jax.ShapeDtypeStruct((NPAD, DH), jnp.float32),
    )(dinv2d, p0[0], p0[1], p1[0], p1[1], xp0, xp1,
      W1, b1.reshape(1, F), W2p)

    p2 = _prop_kernel(e3, yp)

    out = pl.pallas_call(
        _lsm_body,
        grid=(NPAD // _BLK,),
        in_specs=[_row_spec(1), _row_spec(DH), _row_spec(DH), _row_spec(DH),
                  _full_spec(1, C)],
        out_specs=_row_spec(C),
        out_shape=jax.ShapeDtypeStruct((NPAD, C), jnp.float32),
    )(dinv2d, p2[0], p2[1], yp, b2.reshape(1, C))
    return out[:N]


# speedup vs baseline: 30.8239x; 1.0549x over previous
import functools

import jax
import jax.numpy as jnp
from jax import lax
from jax.experimental import pallas as pl
from jax.experimental.pallas import tpu as pltpu
from jax.experimental.pallas import tpu_sc as plsc

N = 10000
E = 320000
F = 128
C = 40
DH = 64
NPAD = 10240
SUB = 125
NSUBROWS = E // SUB
TSUB_PROP = (E // 2) // 16 // SUB
TSUB_DEG = E // 16 // SUB
RPT = NPAD // 16

_MESH = plsc.VectorSubcoreMesh(core_axis_name="c", subcore_axis_name="s")


@functools.partial(
    pl.kernel,
    out_type=jax.ShapeDtypeStruct((NPAD,), jnp.float32),
    mesh=_MESH,
    scratch_types=[
        pltpu.VMEM_SHARED((NPAD,), jnp.float32),
        pltpu.VMEM((TSUB_DEG, SUB), jnp.int32),
        pltpu.VMEM((SUB,), jnp.float32),
        pltpu.VMEM((640,), jnp.float32),
        pltpu.SemaphoreType.DMA,
        pltpu.SemaphoreType.DMA,
    ],
)
def _deg_kernel(e3_hbm, deg_hbm, deg_sh, idx_v, ones_v, stage_v, s_sem0, s_sem1):
    cid = lax.axis_index("c")
    sid = lax.axis_index("s")

    def store16(ref, n, val):
        def body(j, _):
            ref[pl.ds(j * 16, 16)] = jnp.full((16,), val, jnp.float32)
            return 0
        lax.fori_loop(0, n // 16, body, 0)

    store16(ones_v, 112, 1.0)
    ones_v[pl.ds(SUB - 16, 16)] = jnp.full((16,), 1.0, jnp.float32)
    store16(stage_v, 640, 1.0)
    pltpu.sync_copy(stage_v, deg_sh.at[pl.ds(sid * 640, 640)])
    pltpu.sync_copy(e3_hbm.at[1, pl.ds(sid * TSUB_DEG, TSUB_DEG)], idx_v)
    plsc.subcore_barrier()

    SPS = 4
    NSUP = TSUB_DEG // SPS

    def fire(t, sem):
        for k in range(SPS):
            pltpu.async_copy(ones_v, deg_sh.at[idx_v.at[t * SPS + k]], sem,
                             add=True)

    def drain(t, sem):
        for k in range(SPS):
            pltpu.make_async_copy(ones_v, deg_sh.at[idx_v.at[t * SPS + k]],
                                  sem).wait()

    fire(0, s_sem0)
    fire(1, s_sem1)

    def body(i, _):
        t = 2 * i
        drain(t - 2, s_sem0)
        fire(t, s_sem0)
        drain(t - 1, s_sem1)
        fire(t + 1, s_sem1)
        return 0

    lax.fori_loop(1, NSUP // 2, body, 0)
    drain(NSUP - 2, s_sem0)
    drain(NSUP - 1, s_sem1)
    plsc.subcore_barrier()

    base = (sid * 2 + cid) * 320
    pltpu.sync_copy(deg_sh.at[pl.ds(base, 320)], stage_v.at[pl.ds(0, 320)])
    pltpu.sync_copy(stage_v.at[pl.ds(0, 320)], deg_hbm.at[pl.ds(base, 320)])


SPS = 4
NSUP = TSUB_PROP // SPS


@functools.partial(
    pl.kernel,
    out_type=jax.ShapeDtypeStruct((2, NPAD, DH), jnp.float32),
    mesh=_MESH,
    compiler_params=pltpu.CompilerParams(use_tc_tiling_on_sc=False),
    scratch_types=[
        pltpu.VMEM_SHARED((NPAD, DH), jnp.float32),
        pltpu.VMEM((2, TSUB_PROP, SUB), jnp.int32),
        pltpu.VMEM((2, SPS, 128, DH), jnp.float32),
        pltpu.SemaphoreType.DMA,
        pltpu.SemaphoreType.DMA,
        pltpu.SemaphoreType.DMA,
    ],
)
def _prop_kernel(e3_hbm, xp_hbm, out_hbm, acc_sh, idx_v, rows_v,
                 g_sem, s_sem0, s_sem1):
    cid = lax.axis_index("c")
    sid = lax.axis_index("s")
    rbase = sid * RPT
    subbase = cid * (NSUBROWS // 2) + sid * TSUB_PROP
    pltpu.sync_copy(e3_hbm.at[:, pl.ds(subbase, TSUB_PROP)], idx_v)

    zb = rows_v.at[0, 0]

    def zbody(t, _):
        zb[t // 4, pl.ds((t % 4) * 16, 16)] = jnp.zeros((16,), jnp.float32)
        return 0

    lax.fori_loop(0, 512, zbody, 0)
    for j in range(RPT // 128):
        pltpu.sync_copy(zb, acc_sh.at[pl.ds(rbase + j * 128, 128)])
    plsc.subcore_barrier()

    def fire(t, b, sem):
        for k in range(SPS):
            pltpu.async_copy(xp_hbm.at[idx_v.at[0, t * SPS + k]],
                             rows_v.at[b, k, pl.ds(0, SUB)], g_sem)
        for k in range(SPS):
            pltpu.make_async_copy(xp_hbm.at[idx_v.at[0, t * SPS + k]],
                                  rows_v.at[b, k, pl.ds(0, SUB)], g_sem).wait()
        for k in range(SPS):
            pltpu.async_copy(rows_v.at[b, k, pl.ds(0, SUB)],
                             acc_sh.at[idx_v.at[1, t * SPS + k]], sem,
                             add=True)

    def drain(t, b, sem):
        for k in range(SPS):
            pltpu.make_async_copy(rows_v.at[b, k, pl.ds(0, SUB)],
                                  acc_sh.at[idx_v.at[1, t * SPS + k]],
                                  sem).wait()

    fire(0, 0, s_sem0)
    fire(1, 1, s_sem1)

    def body(i, _):
        t = 2 * i
        drain(t - 2, 0, s_sem0)
        fire(t, 0, s_sem0)
        drain(t - 1, 1, s_sem1)
        fire(t + 1, 1, s_sem1)
        return 0

    lax.fori_loop(1, NSUP // 2, body, 0)
    drain(NSUP - 2, 0, s_sem0)
    drain(NSUP - 1, 1, s_sem1)
    plsc.subcore_barrier()

    for j in range(RPT // 128):
        pltpu.sync_copy(acc_sh.at[pl.ds(rbase + j * 128, 128)], zb)
        pltpu.sync_copy(zb, out_hbm.at[cid, pl.ds(rbase + j * 128, 128)])


_BLK = 1024


def _scale_body(deg_ref, x_ref, dinv_ref, xp0_ref, xp1_ref):
    dinv = jax.lax.rsqrt(deg_ref[...])
    dinv_ref[...] = dinv
    xp = dinv * x_ref[...]
    xp0_ref[...] = xp[:, :DH]
    xp1_ref[...] = xp[:, DH:]


def _mlp_body(dinv_ref, pa0_ref, pb0_ref, pa1_ref, pb1_ref, xp0_ref, xp1_ref,
              w1_ref, b1_ref, w2_ref, yp_ref):
    dinv = dinv_ref[...]
    g0 = dinv * (pa0_ref[...] + pb0_ref[...] + xp0_ref[...])
    g1 = dinv * (pa1_ref[...] + pb1_ref[...] + xp1_ref[...])
    g = jnp.concatenate([g0, g1], axis=1)
    h = jnp.maximum(
        jnp.dot(g, w1_ref[...], preferred_element_type=jnp.float32)
        + b1_ref[...], 0.0)
    y = jnp.dot(h, w2_ref[...], preferred_element_type=jnp.float32)
    yp_ref[...] = dinv * y


def _lsm_body(dinv_ref, pa_ref, pb_ref, yp_ref, b2_ref, out_ref):
    logits = (dinv_ref[...] * (pa_ref[...] + pb_ref[...] + yp_ref[...]))[:, :C]
    logits = logits + b2_ref[...]
    m = jnp.max(logits, axis=1, keepdims=True)
    z = logits - m
    lse = jnp.log(jnp.sum(jnp.exp(z), axis=1, keepdims=True))
    out_ref[...] = z - lse


def _row_spec(w):
    return pl.BlockSpec((_BLK, w), lambda i: (i, 0))


def _full_spec(h, w):
    return pl.BlockSpec((h, w), lambda i: (0, 0))


def kernel(x, edge_index, W1, b1, W2, b2):
    e3 = edge_index.reshape(2, NSUBROWS, SUB)
    deg = _deg_kernel(e3)
    deg2d = deg.reshape(NPAD, 1)
    xpad = jnp.pad(x, ((0, NPAD - N), (0, 0)))

    dinv2d, xp0, xp1 = pl.pallas_call(
        _scale_body,
        grid=(NPAD // _BLK,),
        in_specs=[_row_spec(1), _row_spec(F)],
        out_specs=[_row_spec(1), _row_spec(DH), _row_spec(DH)],
        out_shape=[jax.ShapeDtypeStruct((NPAD, 1), jnp.float32),
                   jax.ShapeDtypeStruct((NPAD, DH), jnp.float32),
                   jax.ShapeDtypeStruct((NPAD, DH), jnp.float32)],
    )(deg2d, xpad)

    p0 = _prop_kernel(e3, xp0)
    p1 = _prop_kernel(e3, xp1)

    W2p = jnp.pad(W2, ((0, 0), (0, DH - C)))
    yp = pl.pallas_call(
        _mlp_body,
        grid=(NPAD // _BLK,),
        in_specs=[_row_spec(1)] + [_row_spec(DH)] * 6
        + [_full_spec(F, F), _full_spec(1, F), _full_spec(F, DH)],
        out_specs=_row_spec(DH),
        out_shape=jax.ShapeDtypeStruct((NPAD, DH), jnp.float32),
    )(dinv2d, p0[0], p0[1], p1[0], p1[1], xp0, xp1,
      W1, b1.reshape(1, F), W2p)

    p2 = _prop_kernel(e3, yp)

    out = pl.pallas_call(
        _lsm_body,
        grid=(NPAD // _BLK,),
        in_specs=[_row_spec(1), _row_spec(DH), _row_spec(DH), _row_spec(DH),
                  _full_spec(1, C)],
        out_specs=_row_spec(C),
        out_shape=jax.ShapeDtypeStruct((NPAD, C), jnp.float32),
    )(dinv2d, p2[0], p2[1], yp, b2.reshape(1, C))
    return out[:N]


# speedup vs baseline: 34.7750x; 1.1282x over previous
import functools

import jax
import jax.numpy as jnp
from jax import lax
from jax.experimental import pallas as pl
from jax.experimental.pallas import tpu as pltpu
from jax.experimental.pallas import tpu_sc as plsc

N = 10000
E = 320000
F = 128
C = 40
DH = 64
NPAD = 10240
SUB = 125
NSUBROWS = E // SUB
TSUB_ALL = E // 16 // SUB
TSUB_HALF = TSUB_ALL // 2
RPT = NPAD // 16

_MESH = plsc.VectorSubcoreMesh(core_axis_name="c", subcore_axis_name="s")
_UNTILED = pltpu.CompilerParams(use_tc_tiling_on_sc=False)


def _store16(ref, n, val):
    def body(j, _):
        ref[pl.ds(j * 16, 16)] = jnp.full((16,), val, jnp.float32)
        return 0
    lax.fori_loop(0, n // 16, body, 0)


def _rsqrt16(d):
    y = jnp.full((16,), 2.0**-10 * 1.4142135623730951, jnp.float32)
    for k in range(9, 0, -1):
        y = jnp.where(d < jnp.float32(4.0**k),
                      jnp.float32(2.0**-k * 1.4142135623730951), y)
    for _ in range(6):
        y = y * (jnp.float32(1.5) - jnp.float32(0.5) * d * y * y)
    return y


def _scatter_ones(idx_v, dst_sh, ones_v, sem0, sem1, nsub):
    sps = 4
    nsup = nsub // sps

    def fire(t, sem):
        for k in range(sps):
            pltpu.async_copy(ones_v, dst_sh.at[idx_v.at[1, t * sps + k]], sem,
                             add=True)

    def drain(t, sem):
        for k in range(sps):
            pltpu.make_async_copy(ones_v, dst_sh.at[idx_v.at[1, t * sps + k]],
                                  sem).wait()

    fire(0, sem0)
    fire(1, sem1)

    def body(i, _):
        t = 2 * i
        drain(t - 2, sem0)
        fire(t, sem0)
        drain(t - 1, sem1)
        fire(t + 1, sem1)
        return 0

    lax.fori_loop(1, nsup // 2, body, 0)
    drain(nsup - 2, sem0)
    drain(nsup - 1, sem1)


def _prop_loop(xp_hbm, acc_sh, idx_v, rows_v, g_sem, s_sem0, s_sem1, nsub, sps):
    nsup = nsub // sps

    def fire(t, b, sem):
        for k in range(sps):
            pltpu.async_copy(xp_hbm.at[idx_v.at[0, t * sps + k]],
                             rows_v.at[b, k, pl.ds(0, SUB)], g_sem)
        for k in range(sps):
            pltpu.make_async_copy(xp_hbm.at[idx_v.at[0, t * sps + k]],
                                  rows_v.at[b, k, pl.ds(0, SUB)], g_sem).wait()
        for k in range(sps):
            pltpu.async_copy(rows_v.at[b, k, pl.ds(0, SUB)],
                             acc_sh.at[idx_v.at[1, t * sps + k]], sem,
                             add=True)

    def drain(t, b, sem):
        for k in range(sps):
            pltpu.make_async_copy(rows_v.at[b, k, pl.ds(0, SUB)],
                                  acc_sh.at[idx_v.at[1, t * sps + k]],
                                  sem).wait()

    fire(0, 0, s_sem0)
    fire(1, 1, s_sem1)

    def body(i, _):
        t = 2 * i
        drain(t - 2, 0, s_sem0)
        fire(t, 0, s_sem0)
        drain(t - 1, 1, s_sem1)
        fire(t + 1, 1, s_sem1)
        return 0

    lax.fori_loop(1, nsup // 2, body, 0)
    drain(nsup - 2, 0, s_sem0)
    drain(nsup - 1, 1, s_sem1)


@functools.partial(
    pl.kernel,
    out_type=(jax.ShapeDtypeStruct((NPAD, F), jnp.float32),
              jax.ShapeDtypeStruct((NPAD,), jnp.float32),
              jax.ShapeDtypeStruct((NPAD, DH), jnp.float32),
              jax.ShapeDtypeStruct((NPAD, DH), jnp.float32)),
    mesh=_MESH,
    compiler_params=_UNTILED,
    scratch_types=[
        pltpu.VMEM_SHARED((NPAD,), jnp.float32),
        pltpu.VMEM_SHARED((NPAD, DH), jnp.float32),
        pltpu.VMEM((2, TSUB_ALL, SUB), jnp.int32),
        pltpu.VMEM((2, 2, 128, DH), jnp.float32),
        pltpu.VMEM((SUB,), jnp.float32),
        pltpu.VMEM((RPT,), jnp.float32),
        pltpu.VMEM((RPT,), jnp.float32),
        pltpu.SemaphoreType.DMA,
        pltpu.SemaphoreType.DMA,
        pltpu.SemaphoreType.DMA,
    ],
)
def _mega_kernel(e3_hbm, x_hbm, p1_hbm, dinv_hbm, xp0_hbm, xp1_hbm,
                 deg_sh, acc_sh, idx_v, rows_v, ones_v, stage_v, dstage_v,
                 g_sem, s_sem0, s_sem1):
    cid = lax.axis_index("c")
    sid = lax.axis_index("s")
    rbase = sid * RPT

    pltpu.sync_copy(e3_hbm.at[:, pl.ds(sid * TSUB_ALL, TSUB_ALL)], idx_v)
    _store16(stage_v, RPT, 1.0)
    pltpu.sync_copy(stage_v, deg_sh.at[pl.ds(rbase, RPT)])
    _store16(ones_v, 112, 1.0)
    ones_v[pl.ds(SUB - 16, 16)] = jnp.full((16,), 1.0, jnp.float32)
    plsc.subcore_barrier()

    _scatter_ones(idx_v, deg_sh, ones_v, s_sem0, s_sem1, TSUB_ALL)
    plsc.subcore_barrier()

    pltpu.sync_copy(deg_sh.at[pl.ds(rbase, RPT)], stage_v)

    def rbody(j, _):
        dstage_v[pl.ds(j * 16, 16)] = _rsqrt16(stage_v[pl.ds(j * 16, 16)])
        return 0

    lax.fori_loop(0, RPT // 16, rbody, 0)

    @pl.when(cid == 0)
    def _():
        pltpu.sync_copy(dstage_v, dinv_hbm.at[pl.ds(rbase, RPT)])

    def scale_phase(xp_hbm):
        for j in range(RPT // 128):
            buf = rows_v.at[j % 2, 0]
            pltpu.sync_copy(
                x_hbm.at[pl.ds(rbase + j * 128, 128), pl.ds(cid * DH, DH)], buf)

            def srow(g, _):
                dv16 = dstage_v[pl.ds(j * 128 + g * 16, 16)]
                for i in range(16):
                    dv = dv16[i]
                    r = g * 16 + i
                    for q in range(DH // 16):
                        buf[r, pl.ds(q * 16, 16)] = (
                            buf[r, pl.ds(q * 16, 16)] * dv)
                return 0

            lax.fori_loop(0, 8, srow, 0)
            pltpu.sync_copy(buf, xp_hbm.at[pl.ds(rbase + j * 128, 128)])
            pltpu.sync_copy(buf, acc_sh.at[pl.ds(rbase + j * 128, 128)])

    @pl.when(cid == 0)
    def _():
        scale_phase(xp0_hbm)

    @pl.when(cid == 1)
    def _():
        scale_phase(xp1_hbm)

    plsc.subcore_barrier()

    @pl.when(cid == 0)
    def _():
        _prop_loop(xp0_hbm, acc_sh, idx_v, rows_v, g_sem, s_sem0, s_sem1,
                   TSUB_ALL, 2)

    @pl.when(cid == 1)
    def _():
        _prop_loop(xp1_hbm, acc_sh, idx_v, rows_v, g_sem, s_sem0, s_sem1,
                   TSUB_ALL, 2)

    plsc.subcore_barrier()

    for j in range(RPT // 128):
        buf = rows_v.at[j % 2, 0]
        pltpu.sync_copy(acc_sh.at[pl.ds(rbase + j * 128, 128)], buf)
        pltpu.sync_copy(
            buf, p1_hbm.at[pl.ds(rbase + j * 128, 128), pl.ds(cid * DH, DH)])


@functools.partial(
    pl.kernel,
    out_type=jax.ShapeDtypeStruct((2, NPAD, DH), jnp.float32),
    mesh=_MESH,
    compiler_params=_UNTILED,
    scratch_types=[
        pltpu.VMEM_SHARED((NPAD, DH), jnp.float32),
        pltpu.VMEM((2, TSUB_HALF, SUB), jnp.int32),
        pltpu.VMEM((2, 4, 128, DH), jnp.float32),
        pltpu.SemaphoreType.DMA,
        pltpu.SemaphoreType.DMA,
        pltpu.SemaphoreType.DMA,
    ],
)
def _prop2_kernel(e3_hbm, yp_hbm, out_hbm, acc_sh, idx_v, rows_v,
                  g_sem, s_sem0, s_sem1):
    cid = lax.axis_index("c")
    sid = lax.axis_index("s")
    rbase = sid * RPT
    subbase = cid * (NSUBROWS // 2) + sid * TSUB_HALF
    pltpu.sync_copy(e3_hbm.at[:, pl.ds(subbase, TSUB_HALF)], idx_v)
    zb = rows_v.at[0, 0]

    @pl.when(cid == 0)
    def _():
        for j in range(RPT // 128):
            pltpu.sync_copy(yp_hbm.at[pl.ds(rbase + j * 128, 128)], zb)
            pltpu.sync_copy(zb, acc_sh.at[pl.ds(rbase + j * 128, 128)])

    @pl.when(cid == 1)
    def _():
        def zbody(t, _):
            zb[t // 4, pl.ds((t % 4) * 16, 16)] = jnp.zeros((16,), jnp.float32)
            return 0

        lax.fori_loop(0, 512, zbody, 0)
        for j in range(RPT // 128):
            pltpu.sync_copy(zb, acc_sh.at[pl.ds(rbase + j * 128, 128)])

    plsc.subcore_barrier()
    _prop_loop(yp_hbm, acc_sh, idx_v, rows_v, g_sem, s_sem0, s_sem1,
               TSUB_HALF, 4)
    plsc.subcore_barrier()
    for j in range(RPT // 128):
        pltpu.sync_copy(acc_sh.at[pl.ds(rbase + j * 128, 128)], zb)
        pltpu.sync_copy(zb, out_hbm.at[cid, pl.ds(rbase + j * 128, 128)])


_BLK = 1024


def _mlp_body(dinv_ref, p1_ref, w1_ref, b1_ref, w2_ref, yp_ref):
    dinv = dinv_ref[...]
    g = dinv * p1_ref[...]
    h = jnp.maximum(
        jnp.dot(g, w1_ref[...], preferred_element_type=jnp.float32)
        + b1_ref[...], 0.0)
    y = jnp.dot(h, w2_ref[...], preferred_element_type=jnp.float32)
    yp_ref[...] = dinv * y


def _lsm_body(dinv_ref, pa_ref, pb_ref, b2_ref, out_ref):
    logits = (dinv_ref[...] * (pa_ref[...] + pb_ref[...]))[:, :C]
    logits = logits + b2_ref[...]
    m = jnp.max(logits, axis=1, keepdims=True)
    z = logits - m
    lse = jnp.log(jnp.sum(jnp.exp(z), axis=1, keepdims=True))
    out_ref[...] = z - lse


def _row_spec(w):
    return pl.BlockSpec((_BLK, w), lambda i: (i, 0))


def _full_spec(h, w):
    return pl.BlockSpec((h, w), lambda i: (0, 0))


def kernel(x, edge_index, W1, b1, W2, b2):
    e3 = edge_index.reshape(2, NSUBROWS, SUB)
    xpad = jnp.pad(x, ((0, NPAD - N), (0, 0)))
    p1, dinv, _, _ = _mega_kernel(e3, xpad)
    dinv2d = dinv.reshape(NPAD, 1)

    W2p = jnp.pad(W2, ((0, 0), (0, DH - C)))
    yp = pl.pallas_call(
        _mlp_body,
        grid=(NPAD // _BLK,),
        in_specs=[_row_spec(1), _row_spec(F),
                  _full_spec(F, F), _full_spec(1, F), _full_spec(F, DH)],
        out_specs=_row_spec(DH),
        out_shape=jax.ShapeDtypeStruct((NPAD, DH), jnp.float32),
    )(dinv2d, p1, W1, b1.reshape(1, F), W2p)

    q = _prop2_kernel(e3, yp)

    out = pl.pallas_call(
        _lsm_body,
        grid=(NPAD // _BLK,),
        in_specs=[_row_spec(1), _row_spec(DH), _row_spec(DH),
                  _full_spec(1, C)],
        out_specs=_row_spec(C),
        out_shape=jax.ShapeDtypeStruct((NPAD, C), jnp.float32),
    )(dinv2d, q[0], q[1], b2.reshape(1, C))
    return out[:N]


# speedup vs baseline: 36.7041x; 1.0555x over previous
import functools

import jax
import jax.numpy as jnp
from jax import lax
from jax.experimental import pallas as pl
from jax.experimental.pallas import tpu as pltpu
from jax.experimental.pallas import tpu_sc as plsc

N = 10000
E = 320000
F = 128
C = 40
DH = 64
D2 = 48
NPAD = 10240
SUB = 125
NSUBROWS = E // SUB
TSUB_ALL = E // 16 // SUB
TSUB_HALF = TSUB_ALL // 2
RPT = NPAD // 16

_MESH = plsc.VectorSubcoreMesh(core_axis_name="c", subcore_axis_name="s")
_UNTILED = pltpu.CompilerParams(use_tc_tiling_on_sc=False)


def _store16(ref, n, val):
    def body(j, _):
        ref[pl.ds(j * 16, 16)] = jnp.full((16,), val, jnp.float32)
        return 0
    lax.fori_loop(0, n // 16, body, 0)


def _rsqrt16(d):
    y = jnp.full((16,), 2.0**-10 * 1.4142135623730951, jnp.float32)
    for k in range(9, 0, -1):
        y = jnp.where(d < jnp.float32(4.0**k),
                      jnp.float32(2.0**-k * 1.4142135623730951), y)
    for _ in range(6):
        y = y * (jnp.float32(1.5) - jnp.float32(0.5) * d * y * y)
    return y


def _scatter_ones(idx_v, dst_sh, ones_v, sem0, sem1, nsub):
    sps = 4
    nsup = nsub // sps

    def fire(t, sem):
        for k in range(sps):
            pltpu.async_copy(ones_v, dst_sh.at[idx_v.at[1, t * sps + k]], sem,
                             add=True)

    def drain(t, sem):
        for k in range(sps):
            pltpu.make_async_copy(ones_v, dst_sh.at[idx_v.at[1, t * sps + k]],
                                  sem).wait()

    fire(0, sem0)
    fire(1, sem1)

    def body(i, _):
        t = 2 * i
        drain(t - 2, sem0)
        fire(t, sem0)
        drain(t - 1, sem1)
        fire(t + 1, sem1)
        return 0

    lax.fori_loop(1, nsup // 2, body, 0)
    drain(nsup - 2, sem0)
    drain(nsup - 1, sem1)


def _prop_loop(xp_hbm, acc_sh, idx_v, rows_v, g_sem, s_sem0, s_sem1, nsub, sps):
    nsup = nsub // sps

    def fire(t, b, sem):
        for k in range(sps):
            pltpu.async_copy(xp_hbm.at[idx_v.at[0, t * sps + k]],
                             rows_v.at[b, k, pl.ds(0, SUB)], g_sem)
        for k in range(sps):
            pltpu.make_async_copy(xp_hbm.at[idx_v.at[0, t * sps + k]],
                                  rows_v.at[b, k, pl.ds(0, SUB)], g_sem).wait()
        for k in range(sps):
            pltpu.async_copy(rows_v.at[b, k, pl.ds(0, SUB)],
                             acc_sh.at[idx_v.at[1, t * sps + k]], sem,
                             add=True)

    def drain(t, b, sem):
        for k in range(sps):
            pltpu.make_async_copy(rows_v.at[b, k, pl.ds(0, SUB)],
                                  acc_sh.at[idx_v.at[1, t * sps + k]],
                                  sem).wait()

    fire(0, 0, s_sem0)
    fire(1, 1, s_sem1)

    def body(i, _):
        t = 2 * i
        drain(t - 2, 0, s_sem0)
        fire(t, 0, s_sem0)
        drain(t - 1, 1, s_sem1)
        fire(t + 1, 1, s_sem1)
        return 0

    lax.fori_loop(1, nsup // 2, body, 0)
    drain(nsup - 2, 0, s_sem0)
    drain(nsup - 1, 1, s_sem1)


@functools.partial(
    pl.kernel,
    out_type=(jax.ShapeDtypeStruct((NPAD, F), jnp.float32),
              jax.ShapeDtypeStruct((NPAD,), jnp.float32),
              jax.ShapeDtypeStruct((NPAD, DH), jnp.float32),
              jax.ShapeDtypeStruct((NPAD, DH), jnp.float32)),
    mesh=_MESH,
    compiler_params=_UNTILED,
    scratch_types=[
        pltpu.VMEM_SHARED((NPAD,), jnp.float32),
        pltpu.VMEM_SHARED((NPAD, DH), jnp.float32),
        pltpu.VMEM((2, TSUB_ALL, SUB), jnp.int32),
        pltpu.VMEM((2, 2, 128, DH), jnp.float32),
        pltpu.VMEM((SUB,), jnp.float32),
        pltpu.VMEM((RPT,), jnp.float32),
        pltpu.VMEM((RPT,), jnp.float32),
        pltpu.SemaphoreType.DMA,
        pltpu.SemaphoreType.DMA,
        pltpu.SemaphoreType.DMA,
    ],
)
def _mega_kernel(e3_hbm, x_hbm, p1_hbm, dinv_hbm, xp0_hbm, xp1_hbm,
                 deg_sh, acc_sh, idx_v, rows_v, ones_v, stage_v, dstage_v,
                 g_sem, s_sem0, s_sem1):
    cid = lax.axis_index("c")
    sid = lax.axis_index("s")
    rbase = sid * RPT

    pltpu.sync_copy(e3_hbm.at[:, pl.ds(sid * TSUB_ALL, TSUB_ALL)], idx_v)
    _store16(stage_v, RPT, 1.0)
    pltpu.sync_copy(stage_v, deg_sh.at[pl.ds(rbase, RPT)])
    _store16(ones_v, 112, 1.0)
    ones_v[pl.ds(SUB - 16, 16)] = jnp.full((16,), 1.0, jnp.float32)
    plsc.subcore_barrier()

    _scatter_ones(idx_v, deg_sh, ones_v, s_sem0, s_sem1, TSUB_ALL)
    plsc.subcore_barrier()

    pltpu.sync_copy(deg_sh.at[pl.ds(rbase, RPT)], stage_v)

    def rbody(j, _):
        dstage_v[pl.ds(j * 16, 16)] = _rsqrt16(stage_v[pl.ds(j * 16, 16)])
        return 0

    lax.fori_loop(0, RPT // 16, rbody, 0)

    @pl.when(cid == 0)
    def _():
        pltpu.sync_copy(dstage_v, dinv_hbm.at[pl.ds(rbase, RPT)])

    def scale_phase(xp_hbm):
        for j in range(RPT // 128):
            buf = rows_v.at[j % 2, 0]
            r0 = rbase + j * 128

            @pl.when(r0 + 128 <= N)
            def _():
                pltpu.sync_copy(
                    x_hbm.at[pl.ds(r0, 128), pl.ds(cid * DH, DH)], buf)

            @pl.when(r0 + 128 > N)
            def _():
                def zbody(t, _):
                    buf[t // 4, pl.ds((t % 4) * 16, 16)] = (
                        jnp.zeros((16,), jnp.float32))
                    return 0

                lax.fori_loop(0, 512, zbody, 0)

                @pl.when(r0 < N)
                def _():
                    pltpu.sync_copy(
                        x_hbm.at[pl.ds(r0, N % 128), pl.ds(cid * DH, DH)],
                        buf.at[pl.ds(0, N % 128)])

            def srow(g, _):
                dv16 = dstage_v[pl.ds(j * 128 + g * 16, 16)]
                for i in range(16):
                    dv = dv16[i]
                    r = g * 16 + i
                    for q in range(DH // 16):
                        buf[r, pl.ds(q * 16, 16)] = (
                            buf[r, pl.ds(q * 16, 16)] * dv)
                return 0

            lax.fori_loop(0, 8, srow, 0)
            pltpu.sync_copy(buf, xp_hbm.at[pl.ds(rbase + j * 128, 128)])
            pltpu.sync_copy(buf, acc_sh.at[pl.ds(rbase + j * 128, 128)])

    @pl.when(cid == 0)
    def _():
        scale_phase(xp0_hbm)

    @pl.when(cid == 1)
    def _():
        scale_phase(xp1_hbm)

    plsc.subcore_barrier()

    @pl.when(cid == 0)
    def _():
        _prop_loop(xp0_hbm, acc_sh, idx_v, rows_v, g_sem, s_sem0, s_sem1,
                   TSUB_ALL, 2)

    @pl.when(cid == 1)
    def _():
        _prop_loop(xp1_hbm, acc_sh, idx_v, rows_v, g_sem, s_sem0, s_sem1,
                   TSUB_ALL, 2)

    plsc.subcore_barrier()

    for j in range(RPT // 128):
        buf = rows_v.at[j % 2, 0]
        pltpu.sync_copy(acc_sh.at[pl.ds(rbase + j * 128, 128)], buf)
        pltpu.sync_copy(
            buf, p1_hbm.at[pl.ds(rbase + j * 128, 128), pl.ds(cid * DH, DH)])


@functools.partial(
    pl.kernel,
    out_type=jax.ShapeDtypeStruct((2, NPAD, D2), jnp.float32),
    mesh=_MESH,
    compiler_params=_UNTILED,
    scratch_types=[
        pltpu.VMEM_SHARED((NPAD, D2), jnp.float32),
        pltpu.VMEM((2, TSUB_HALF, SUB), jnp.int32),
        pltpu.VMEM((2, 4, 128, D2), jnp.float32),
        pltpu.SemaphoreType.DMA,
        pltpu.SemaphoreType.DMA,
        pltpu.SemaphoreType.DMA,
    ],
)
def _prop2_kernel(e3_hbm, yp_hbm, out_hbm, acc_sh, idx_v, rows_v,
                  g_sem, s_sem0, s_sem1):
    cid = lax.axis_index("c")
    sid = lax.axis_index("s")
    rbase = sid * RPT
    subbase = cid * (NSUBROWS // 2) + sid * TSUB_HALF
    pltpu.sync_copy(e3_hbm.at[:, pl.ds(subbase, TSUB_HALF)], idx_v)
    zb = rows_v.at[0, 0]

    @pl.when(cid == 0)
    def _():
        for j in range(RPT // 128):
            pltpu.sync_copy(yp_hbm.at[pl.ds(rbase + j * 128, 128)], zb)
            pltpu.sync_copy(zb, acc_sh.at[pl.ds(rbase + j * 128, 128)])

    @pl.when(cid == 1)
    def _():
        def zbody(t, _):
            zb[t // 3, pl.ds((t % 3) * 16, 16)] = jnp.zeros((16,), jnp.float32)
            return 0

        lax.fori_loop(0, 128 * (D2 // 16), zbody, 0)
        for j in range(RPT // 128):
            pltpu.sync_copy(zb, acc_sh.at[pl.ds(rbase + j * 128, 128)])

    plsc.subcore_barrier()
    _prop_loop(yp_hbm, acc_sh, idx_v, rows_v, g_sem, s_sem0, s_sem1,
               TSUB_HALF, 4)
    plsc.subcore_barrier()
    for j in range(RPT // 128):
        pltpu.sync_copy(acc_sh.at[pl.ds(rbase + j * 128, 128)], zb)
        pltpu.sync_copy(zb, out_hbm.at[cid, pl.ds(rbase + j * 128, 128)])


_BLK = 1024


def _mlp_body(dinv_ref, p1_ref, w1_ref, b1_ref, w2_ref, yp_ref):
    dinv = dinv_ref[...]
    g = dinv * p1_ref[...]
    h = jnp.maximum(
        jnp.dot(g, w1_ref[...], preferred_element_type=jnp.float32)
        + b1_ref[...], 0.0)
    y = jnp.dot(h, w2_ref[...], preferred_element_type=jnp.float32)
    yp_ref[...] = dinv * y


def _lsm_body(dinv_ref, pa_ref, pb_ref, b2_ref, out_ref):
    logits = (dinv_ref[...] * (pa_ref[...] + pb_ref[...]))[:, :C]
    logits = logits + b2_ref[...]
    m = jnp.max(logits, axis=1, keepdims=True)
    z = logits - m
    lse = jnp.log(jnp.sum(jnp.exp(z), axis=1, keepdims=True))
    out_ref[...] = z - lse


def _row_spec(w):
    return pl.BlockSpec((_BLK, w), lambda i: (i, 0))


def _full_spec(h, w):
    return pl.BlockSpec((h, w), lambda i: (0, 0))


def kernel(x, edge_index, W1, b1, W2, b2):
    e3 = edge_index.reshape(2, NSUBROWS, SUB)
    p1, dinv, _, _ = _mega_kernel(e3, x)
    dinv2d = dinv.reshape(NPAD, 1)

    W2p = jnp.pad(W2, ((0, 0), (0, D2 - C)))
    yp = pl.pallas_call(
        _mlp_body,
        grid=(NPAD // _BLK,),
        in_specs=[_row_spec(1), _row_spec(F),
                  _full_spec(F, F), _full_spec(1, F), _full_spec(F, D2)],
        out_specs=_row_spec(D2),
        out_shape=jax.ShapeDtypeStruct((NPAD, D2), jnp.float32),
    )(dinv2d, p1, W1, b1.reshape(1, F), W2p)

    q = _prop2_kernel(e3, yp)

    out = pl.pallas_call(
        _lsm_body,
        grid=(NPAD // _BLK,),
        in_specs=[_row_spec(1), _row_spec(D2), _row_spec(D2),
                  _full_spec(1, C)],
        out_specs=_row_spec(C),
        out_shape=jax.ShapeDtypeStruct((N, C), jnp.float32),
    )(dinv2d, q[0], q[1], b2.reshape(1, C))
    return out


# speedup vs baseline: 36.8010x; 1.0026x over previous
import functools

import jax
import jax.numpy as jnp
from jax import lax
from jax.experimental import pallas as pl
from jax.experimental.pallas import tpu as pltpu
from jax.experimental.pallas import tpu_sc as plsc

N = 10000
E = 320000
F = 128
C = 40
DH = 64
D2 = 48
NPAD = 10240
SUB = 125
NSUBROWS = E // SUB
TSUB_ALL = E // 16 // SUB
TSUB_HALF = TSUB_ALL // 2
RPT = NPAD // 16

_MESH = plsc.VectorSubcoreMesh(core_axis_name="c", subcore_axis_name="s")
_UNTILED = pltpu.CompilerParams(use_tc_tiling_on_sc=False)


def _store16(ref, n, val):
    def body(j, _):
        ref[pl.ds(j * 16, 16)] = jnp.full((16,), val, jnp.float32)
        return 0
    lax.fori_loop(0, n // 16, body, 0)


def _rsqrt16(d):
    y = jnp.full((16,), 2.0**-10 * 1.4142135623730951, jnp.float32)
    for k in range(9, 0, -1):
        y = jnp.where(d < jnp.float32(4.0**k),
                      jnp.float32(2.0**-k * 1.4142135623730951), y)
    for _ in range(6):
        y = y * (jnp.float32(1.5) - jnp.float32(0.5) * d * y * y)
    return y


def _scatter_ones(idx_v, dst_sh, ones_v, sem0, sem1, nsub):
    sps = 4
    nsup = nsub // sps

    def fire(t, sem):
        for k in range(sps):
            pltpu.async_copy(ones_v, dst_sh.at[idx_v.at[1, t * sps + k]], sem,
                             add=True)

    def drain(t, sem):
        for k in range(sps):
            pltpu.make_async_copy(ones_v, dst_sh.at[idx_v.at[1, t * sps + k]],
                                  sem).wait()

    fire(0, sem0)
    fire(1, sem1)

    def body(i, _):
        t = 2 * i
        drain(t - 2, sem0)
        fire(t, sem0)
        drain(t - 1, sem1)
        fire(t + 1, sem1)
        return 0

    lax.fori_loop(1, nsup // 2, body, 0)
    drain(nsup - 2, sem0)
    drain(nsup - 1, sem1)


def _prop_loop(xp_hbm, acc_sh, idx_v, rows_v, g_sem, s_sem0, s_sem1, nsub, sps):
    nsup = nsub // sps

    def fire(t, b, sem):
        for k in range(sps):
            pltpu.async_copy(xp_hbm.at[idx_v.at[0, t * sps + k]],
                             rows_v.at[b, k, pl.ds(0, SUB)], g_sem)
        for k in range(sps):
            pltpu.make_async_copy(xp_hbm.at[idx_v.at[0, t * sps + k]],
                                  rows_v.at[b, k, pl.ds(0, SUB)], g_sem).wait()
            pltpu.async_copy(rows_v.at[b, k, pl.ds(0, SUB)],
                             acc_sh.at[idx_v.at[1, t * sps + k]], sem,
                             add=True)

    def drain(t, b, sem):
        for k in range(sps):
            pltpu.make_async_copy(rows_v.at[b, k, pl.ds(0, SUB)],
                                  acc_sh.at[idx_v.at[1, t * sps + k]],
                                  sem).wait()

    fire(0, 0, s_sem0)
    fire(1, 1, s_sem1)

    def body(i, _):
        t = 2 * i
        drain(t - 2, 0, s_sem0)
        fire(t, 0, s_sem0)
        drain(t - 1, 1, s_sem1)
        fire(t + 1, 1, s_sem1)
        return 0

    lax.fori_loop(1, nsup // 2, body, 0)
    drain(nsup - 2, 0, s_sem0)
    drain(nsup - 1, 1, s_sem1)


@functools.partial(
    pl.kernel,
    out_type=(jax.ShapeDtypeStruct((NPAD, F), jnp.float32),
              jax.ShapeDtypeStruct((NPAD,), jnp.float32),
              jax.ShapeDtypeStruct((NPAD, DH), jnp.float32),
              jax.ShapeDtypeStruct((NPAD, DH), jnp.float32)),
    mesh=_MESH,
    compiler_params=_UNTILED,
    scratch_types=[
        pltpu.VMEM_SHARED((NPAD,), jnp.float32),
        pltpu.VMEM_SHARED((NPAD, DH), jnp.float32),
        pltpu.VMEM((2, TSUB_ALL, SUB), jnp.int32),
        pltpu.VMEM((2, 2, 128, DH), jnp.float32),
        pltpu.VMEM((SUB,), jnp.float32),
        pltpu.VMEM((RPT,), jnp.float32),
        pltpu.VMEM((RPT,), jnp.float32),
        pltpu.SemaphoreType.DMA,
        pltpu.SemaphoreType.DMA,
        pltpu.SemaphoreType.DMA,
    ],
)
def _mega_kernel(e3_hbm, x_hbm, p1_hbm, dinv_hbm, xp0_hbm, xp1_hbm,
                 deg_sh, acc_sh, idx_v, rows_v, ones_v, stage_v, dstage_v,
                 g_sem, s_sem0, s_sem1):
    cid = lax.axis_index("c")
    sid = lax.axis_index("s")
    rbase = sid * RPT

    pltpu.sync_copy(e3_hbm.at[:, pl.ds(sid * TSUB_ALL, TSUB_ALL)], idx_v)
    _store16(stage_v, RPT, 1.0)
    pltpu.sync_copy(stage_v, deg_sh.at[pl.ds(rbase, RPT)])
    _store16(ones_v, 112, 1.0)
    ones_v[pl.ds(SUB - 16, 16)] = jnp.full((16,), 1.0, jnp.float32)
    plsc.subcore_barrier()

    _scatter_ones(idx_v, deg_sh, ones_v, s_sem0, s_sem1, TSUB_ALL)
    plsc.subcore_barrier()

    pltpu.sync_copy(deg_sh.at[pl.ds(rbase, RPT)], stage_v)

    def rbody(j, _):
        dstage_v[pl.ds(j * 16, 16)] = _rsqrt16(stage_v[pl.ds(j * 16, 16)])
        return 0

    lax.fori_loop(0, RPT // 16, rbody, 0)

    @pl.when(cid == 0)
    def _():
        pltpu.sync_copy(dstage_v, dinv_hbm.at[pl.ds(rbase, RPT)])

    def scale_phase(xp_hbm):
        for j in range(RPT // 128):
            buf = rows_v.at[j % 2, 0]
            r0 = rbase + j * 128

            @pl.when(r0 + 128 <= N)
            def _():
                pltpu.sync_copy(
                    x_hbm.at[pl.ds(r0, 128), pl.ds(cid * DH, DH)], buf)

            @pl.when(r0 + 128 > N)
            def _():
                def zbody(t, _):
                    buf[t // 4, pl.ds((t % 4) * 16, 16)] = (
                        jnp.zeros((16,), jnp.float32))
                    return 0

                lax.fori_loop(0, 512, zbody, 0)

                @pl.when(r0 < N)
                def _():
                    pltpu.sync_copy(
                        x_hbm.at[pl.ds(r0, N % 128), pl.ds(cid * DH, DH)],
                        buf.at[pl.ds(0, N % 128)])

            def srow(g, _):
                dv16 = dstage_v[pl.ds(j * 128 + g * 16, 16)]
                for i in range(16):
                    dv = dv16[i]
                    r = g * 16 + i
                    for q in range(DH // 16):
                        buf[r, pl.ds(q * 16, 16)] = (
                            buf[r, pl.ds(q * 16, 16)] * dv)
                return 0

            lax.fori_loop(0, 8, srow, 0)
            pltpu.sync_copy(buf, xp_hbm.at[pl.ds(rbase + j * 128, 128)])
            pltpu.sync_copy(buf, acc_sh.at[pl.ds(rbase + j * 128, 128)])

    @pl.when(cid == 0)
    def _():
        scale_phase(xp0_hbm)

    @pl.when(cid == 1)
    def _():
        scale_phase(xp1_hbm)

    plsc.subcore_barrier()

    @pl.when(cid == 0)
    def _():
        _prop_loop(xp0_hbm, acc_sh, idx_v, rows_v, g_sem, s_sem0, s_sem1,
                   TSUB_ALL, 2)

    @pl.when(cid == 1)
    def _():
        _prop_loop(xp1_hbm, acc_sh, idx_v, rows_v, g_sem, s_sem0, s_sem1,
                   TSUB_ALL, 2)

    plsc.subcore_barrier()

    for j in range(RPT // 128):
        buf = rows_v.at[j % 2, 0]
        pltpu.sync_copy(acc_sh.at[pl.ds(rbase + j * 128, 128)], buf)
        pltpu.sync_copy(
            buf, p1_hbm.at[pl.ds(rbase + j * 128, 128), pl.ds(cid * DH, DH)])


@functools.partial(
    pl.kernel,
    out_type=jax.ShapeDtypeStruct((2, NPAD, D2), jnp.float32),
    mesh=_MESH,
    compiler_params=_UNTILED,
    scratch_types=[
        pltpu.VMEM_SHARED((NPAD, D2), jnp.float32),
        pltpu.VMEM((2, TSUB_HALF, SUB), jnp.int32),
        pltpu.VMEM((2, 4, 128, D2), jnp.float32),
        pltpu.SemaphoreType.DMA,
        pltpu.SemaphoreType.DMA,
        pltpu.SemaphoreType.DMA,
    ],
)
def _prop2_kernel(e3_hbm, yp_hbm, out_hbm, acc_sh, idx_v, rows_v,
                  g_sem, s_sem0, s_sem1):
    cid = lax.axis_index("c")
    sid = lax.axis_index("s")
    rbase = sid * RPT
    subbase = cid * (NSUBROWS // 2) + sid * TSUB_HALF
    pltpu.sync_copy(e3_hbm.at[:, pl.ds(subbase, TSUB_HALF)], idx_v)
    zb = rows_v.at[0, 0]

    @pl.when(cid == 0)
    def _():
        for j in range(RPT // 128):
            pltpu.sync_copy(yp_hbm.at[pl.ds(rbase + j * 128, 128)], zb)
            pltpu.sync_copy(zb, acc_sh.at[pl.ds(rbase + j * 128, 128)])

    @pl.when(cid == 1)
    def _():
        def zbody(t, _):
            zb[t // 3, pl.ds((t % 3) * 16, 16)] = jnp.zeros((16,), jnp.float32)
            return 0

        lax.fori_loop(0, 128 * (D2 // 16), zbody, 0)
        for j in range(RPT // 128):
            pltpu.sync_copy(zb, acc_sh.at[pl.ds(rbase + j * 128, 128)])

    plsc.subcore_barrier()
    _prop_loop(yp_hbm, acc_sh, idx_v, rows_v, g_sem, s_sem0, s_sem1,
               TSUB_HALF, 4)
    plsc.subcore_barrier()
    for j in range(RPT // 128):
        pltpu.sync_copy(acc_sh.at[pl.ds(rbase + j * 128, 128)], zb)
        pltpu.sync_copy(zb, out_hbm.at[cid, pl.ds(rbase + j * 128, 128)])


_BLK = 1024


def _mlp_body(dinv_ref, p1_ref, w1_ref, b1_ref, w2_ref, yp_ref):
    dinv = dinv_ref[...]
    g = dinv * p1_ref[...]
    h = jnp.maximum(
        jnp.dot(g, w1_ref[...], preferred_element_type=jnp.float32)
        + b1_ref[...], 0.0)
    y = jnp.dot(h, w2_ref[...], preferred_element_type=jnp.float32)
    yp_ref[...] = dinv * y


def _lsm_body(dinv_ref, pa_ref, pb_ref, b2_ref, out_ref):
    logits = (dinv_ref[...] * (pa_ref[...] + pb_ref[...]))[:, :C]
    logits = logits + b2_ref[...]
    m = jnp.max(logits, axis=1, keepdims=True)
    z = logits - m
    lse = jnp.log(jnp.sum(jnp.exp(z), axis=1, keepdims=True))
    out_ref[...] = z - lse


def _row_spec(w):
    return pl.BlockSpec((_BLK, w), lambda i: (i, 0))


def _full_spec(h, w):
    return pl.BlockSpec((h, w), lambda i: (0, 0))


def kernel(x, edge_index, W1, b1, W2, b2):
    e3 = edge_index.reshape(2, NSUBROWS, SUB)
    p1, dinv, _, _ = _mega_kernel(e3, x)
    dinv2d = dinv.reshape(NPAD, 1)

    W2p = jnp.pad(W2, ((0, 0), (0, D2 - C)))
    yp = pl.pallas_call(
        _mlp_body,
        grid=(NPAD // _BLK,),
        in_specs=[_row_spec(1), _row_spec(F),
                  _full_spec(F, F), _full_spec(1, F), _full_spec(F, D2)],
        out_specs=_row_spec(D2),
        out_shape=jax.ShapeDtypeStruct((NPAD, D2), jnp.float32),
    )(dinv2d, p1, W1, b1.reshape(1, F), W2p)

    q = _prop2_kernel(e3, yp)

    out = pl.pallas_call(
        _lsm_body,
        grid=(NPAD // _BLK,),
        in_specs=[_row_spec(1), _row_spec(D2), _row_spec(D2),
                  _full_spec(1, C)],
        out_specs=_row_spec(C),
        out_shape=jax.ShapeDtypeStruct((N, C), jnp.float32),
    )(dinv2d, q[0], q[1], b2.reshape(1, C))
    return out
